# Initial kernel scaffold; baseline (speedup 1.0000x reference)
import jax
import jax.numpy as jnp
from jax.experimental import pallas as pl


def kernel(inputs):
    raise NotImplementedError("write your pallas kernel here")


# speedup vs baseline: 1.0263x; 1.0263x over previous
import jax
import jax.numpy as jnp
from jax.experimental import pallas as pl


def _body(x_ref, o_ref):
    x = x_ref[0]
    m = jnp.max(jnp.abs(x), axis=1)
    cnt = jnp.sum((m > 0).astype(jnp.int32))
    idx = jnp.maximum(cnt - 1, 0)
    o_ref[0, :, :] = x_ref[0, pl.ds(idx, 1), :]


def kernel(inputs):
    B, S, D = inputs.shape
    out = pl.pallas_call(
        _body,
        grid=(B,),
        in_specs=[pl.BlockSpec((1, S, D), lambda b: (b, 0, 0))],
        out_specs=pl.BlockSpec((1, 1, D), lambda b: (b, 0, 0)),
        out_shape=jax.ShapeDtypeStruct((B, 1, D), inputs.dtype),
    )(inputs)
    return out.reshape(B, D)
